# Initial kernel scaffold; baseline (speedup 1.0000x reference)
import jax
import jax.numpy as jnp
from jax.experimental import pallas as pl


def kernel(x, edge_index, edge_attr, w0a, b0a, w0b, b0b, w1a, b1a, w1b, b1b, w2a, b2a, w2b, b2b, wf, bf):
    raise NotImplementedError("write your pallas kernel here")


# speedup vs baseline: 2.9759x; 2.9759x over previous
import functools

import jax
import jax.numpy as jnp
from jax import lax
from jax.experimental import pallas as pl
from jax.experimental.pallas import tpu as pltpu
from jax.experimental.pallas import tpu_sc as plsc

N = 10000
E = 320000
D = 128
H = 64
OUT = 4

NC = 2
NS = 16
NW = NC * NS
L = 16

NR = 320
NPAD = NW * NR
EPW = E // NW
CB = 80
NBLK = EPW // CB
ROWS_PER_TILE = NPAD // NS

CAP = 12800
SCHUNK = 20000
GB = 128
DBASE = CAP + GB + 16

_mesh = plsc.VectorSubcoreMesh(core_axis_name="c", subcore_axis_name="s")


def _wid():
    return lax.axis_index("s") * NC + lax.axis_index("c")


def _ew_add_relu(a_rows, b_rows):
    def body(i, _):
        r = i >> 2
        col = (i & 3) * L
        av = a_rows[r, pl.ds(col, L)]
        bv = b_rows[r, pl.ds(col, L)]
        a_rows[r, pl.ds(col, L)] = jnp.maximum(av + bv, 0.0)
        return 0

    lax.fori_loop(0, CB * H // L, body, 0)


def _sc_edge0(a_hbm, b_hbm, dsti_hbm, srci_hbm, zeros16_hbm, t0_hbm, cntp_hbm,
              dstc, srcc, a_rows, b_rows, ones_v, cacc):
    c = lax.axis_index("c")
    s = lax.axis_index("s")
    wid = _wid()

    pltpu.sync_copy(dsti_hbm.at[wid], dstc)
    pltpu.sync_copy(srci_hbm.at[wid], srcc)
    pltpu.sync_copy(zeros16_hbm.at[pl.ds(s * ROWS_PER_TILE, ROWS_PER_TILE)],
                    cacc.at[pl.ds(s * ROWS_PER_TILE, ROWS_PER_TILE)])

    def fill_ones(i, _):
        ones_v[i, pl.ds(0, L)] = jnp.ones((L,), jnp.float32)
        return 0

    lax.fori_loop(0, CB, fill_ones, 0)
    plsc.subcore_barrier()

    def blk(j, _):
        pltpu.sync_copy(a_hbm.at[dstc.at[j]], a_rows)
        pltpu.sync_copy(b_hbm.at[srcc.at[j]], b_rows)
        _ew_add_relu(a_rows, b_rows)
        pltpu.sync_copy(a_rows, t0_hbm.at[pl.ds(wid * EPW + j * CB, CB)])
        pltpu.sync_copy(ones_v, cacc.at[dstc.at[j]], add=True)
        return 0

    lax.fori_loop(0, NBLK, blk, 0)
    plsc.subcore_barrier()
    pltpu.sync_copy(cacc.at[pl.ds(s * ROWS_PER_TILE, ROWS_PER_TILE)],
                    cntp_hbm.at[c, pl.ds(s * ROWS_PER_TILE, ROWS_PER_TILE)])


def _sc_segmax(u_hbm, dstf_hbm, zeros_hbm, h0_hbm,
               accv, dchunk, elist, urows):
    wid = _wid()
    lo = wid * NR
    hi = lo + NR

    pltpu.sync_copy(zeros_hbm.at[pl.ds(0, NR)], accv)

    iota16 = lax.iota(jnp.int32, L)

    def chunk(cix, off):
        pltpu.sync_copy(dstf_hbm.at[pl.ds(cix * SCHUNK, SCHUNK)], dchunk)

        def scan16(i, off):
            dv = dchunk[pl.ds(i * L, L)]
            m = (dv >= lo) & (dv < hi)
            mi = jnp.where(m, 1, 0)
            pos = off + jnp.cumsum(mi) - 1
            eid = iota16 + (cix * SCHUNK + i * L)
            plsc.store_scatter(elist, [pos], eid, mask=m)
            plsc.store_scatter(elist, [pos + DBASE], dv, mask=m)
            return off + jnp.max(jnp.cumsum(mi))

        return lax.fori_loop(0, SCHUNK // L, scan16, off)

    off = lax.fori_loop(0, E // SCHUNK, chunk, jnp.int32(0))

    def padz(i, _):
        pos = off + iota16 + i * L
        plsc.store_scatter(elist, [pos], jnp.zeros((L,), jnp.int32))
        return 0

    lax.fori_loop(0, GB // L + 1, padz, 0)

    nblks = (off + GB - 1) // GB

    def gblk(b, _):
        pltpu.sync_copy(u_hbm.at[elist.at[pl.ds(b * GB, GB)]], urows)

        def emax(q, _):
            dvec = elist[pl.ds(DBASE + b * GB + q * L, L)]
            for k2 in range(L):
                k = q * L + k2

                @pl.when(b * GB + k < off)
                def _():
                    dl = dvec[k2] - lo
                    for jj in range(H // L):
                        av = accv[dl, pl.ds(jj * L, L)]
                        uv = urows[k, pl.ds(jj * L, L)]
                        accv[dl, pl.ds(jj * L, L)] = jnp.maximum(av, uv)

            return 0

        lax.fori_loop(0, GB // L, emax, 0)
        return 0

    lax.fori_loop(0, nblks, gblk, 0)
    pltpu.sync_copy(accv, h0_hbm.at[pl.ds(lo, NR)])


def _sc_edge_mean(a_hbm, b_hbm, dsti_hbm, srci_hbm, zeros_hbm, sump_hbm,
                  dstc, srcc, a_rows, b_rows, facc):
    c = lax.axis_index("c")
    s = lax.axis_index("s")
    wid = _wid()

    pltpu.sync_copy(dsti_hbm.at[wid], dstc)
    pltpu.sync_copy(srci_hbm.at[wid], srcc)
    pltpu.sync_copy(zeros_hbm.at[pl.ds(s * ROWS_PER_TILE, ROWS_PER_TILE)],
                    facc.at[pl.ds(s * ROWS_PER_TILE, ROWS_PER_TILE)])
    plsc.subcore_barrier()

    def blk(j, _):
        pltpu.sync_copy(a_hbm.at[dstc.at[j]], a_rows)
        pltpu.sync_copy(b_hbm.at[srcc.at[j]], b_rows)
        _ew_add_relu(a_rows, b_rows)
        pltpu.sync_copy(a_rows, facc.at[dstc.at[j]], add=True)
        return 0

    lax.fori_loop(0, NBLK, blk, 0)
    plsc.subcore_barrier()
    pltpu.sync_copy(facc.at[pl.ds(s * ROWS_PER_TILE, ROWS_PER_TILE)],
                    sump_hbm.at[c, pl.ds(s * ROWS_PER_TILE, ROWS_PER_TILE)])


def _make_edge0():
    return pl.kernel(
        _sc_edge0,
        out_type=(jax.ShapeDtypeStruct((E, H), jnp.float32),
                  jax.ShapeDtypeStruct((NC, NPAD, L), jnp.float32)),
        mesh=_mesh,
        compiler_params=pltpu.CompilerParams(use_tc_tiling_on_sc=False, needs_layout_passes=False),
        scratch_types=[
            pltpu.VMEM((NBLK, CB), jnp.int32),
            pltpu.VMEM((NBLK, CB), jnp.int32),
            pltpu.VMEM((CB, H), jnp.float32),
            pltpu.VMEM((CB, H), jnp.float32),
            pltpu.VMEM((CB, L), jnp.float32),
            pltpu.VMEM_SHARED((NPAD, L), jnp.float32),
        ],
    )


def _make_segmax():
    return pl.kernel(
        _sc_segmax,
        out_type=jax.ShapeDtypeStruct((NPAD, H), jnp.float32),
        mesh=_mesh,
        compiler_params=pltpu.CompilerParams(use_tc_tiling_on_sc=False, needs_layout_passes=False),
        scratch_types=[
            pltpu.VMEM((NR, H), jnp.float32),
            pltpu.VMEM((SCHUNK,), jnp.int32),
            pltpu.VMEM((DBASE + CAP + GB + 16,), jnp.int32),
            pltpu.VMEM((GB, H), jnp.float32),
        ],
    )


def _make_edge_mean():
    return pl.kernel(
        _sc_edge_mean,
        out_type=jax.ShapeDtypeStruct((NC, NPAD, H), jnp.float32),
        mesh=_mesh,
        compiler_params=pltpu.CompilerParams(use_tc_tiling_on_sc=False, needs_layout_passes=False),
        scratch_types=[
            pltpu.VMEM((NBLK, CB), jnp.int32),
            pltpu.VMEM((NBLK, CB), jnp.int32),
            pltpu.VMEM((CB, H), jnp.float32),
            pltpu.VMEM((CB, H), jnp.float32),
            pltpu.VMEM_SHARED((NPAD, H), jnp.float32),
        ],
    )


def _tc_ab_body(din, x_ref, wa_ref, ba_ref, a_ref, b_ref):
    xv = x_ref[...]
    wtop = wa_ref[pl.ds(0, din), :]
    wbot = wa_ref[pl.ds(din, din), :]
    b_ref[...] = jnp.dot(xv, wbot, preferred_element_type=jnp.float32, precision=lax.Precision.HIGHEST)
    a_ref[...] = (jnp.dot(xv, wtop - wbot, preferred_element_type=jnp.float32, precision=lax.Precision.HIGHEST)
                  + ba_ref[...])


def _tc_ab(x, wa, ba, din):
    body = functools.partial(_tc_ab_body, din)
    return pl.pallas_call(
        body,
        out_shape=(jax.ShapeDtypeStruct((NPAD, H), jnp.float32),
                   jax.ShapeDtypeStruct((NPAD, H), jnp.float32)),
    )(x, wa, ba)


UB = 4000


def _tc_u_body(t_ref, w_ref, b_ref, u_ref):
    u_ref[...] = (jnp.dot(t_ref[...], w_ref[...],
                          preferred_element_type=jnp.float32, precision=lax.Precision.HIGHEST) + b_ref[...])


def _tc_u(t0, w, b):
    return pl.pallas_call(
        _tc_u_body,
        grid=(E // UB,),
        in_specs=[
            pl.BlockSpec((UB, H), lambda i: (i, 0)),
            pl.BlockSpec((H, H), lambda i: (0, 0)),
            pl.BlockSpec((H,), lambda i: (0,)),
        ],
        out_specs=pl.BlockSpec((UB, H), lambda i: (i, 0)),
        out_shape=jax.ShapeDtypeStruct((E, H), jnp.float32),
    )(t0, w, b)


def _mean_h(p_ref, cp_ref, wb_ref, bb_ref):
    sums = p_ref[0] + p_ref[1]
    cnt = cp_ref[0, :, pl.ds(0, 1)] + cp_ref[1, :, pl.ds(0, 1)]
    mean = sums / jnp.maximum(cnt, 1.0)
    h = jnp.dot(mean, wb_ref[...], preferred_element_type=jnp.float32, precision=lax.Precision.HIGHEST) + bb_ref[...]
    return jnp.where(cnt > 0.0, h, 0.0)


def _tc_mid_body(p_ref, cp_ref, wb_ref, bb_ref, wa_ref, ba_ref, a_ref, b_ref):
    h = jnp.maximum(_mean_h(p_ref, cp_ref, wb_ref, bb_ref), 0.0)
    wtop = wa_ref[pl.ds(0, H), :]
    wbot = wa_ref[pl.ds(H, H), :]
    b_ref[...] = jnp.dot(h, wbot, preferred_element_type=jnp.float32, precision=lax.Precision.HIGHEST)
    a_ref[...] = (jnp.dot(h, wtop - wbot, preferred_element_type=jnp.float32, precision=lax.Precision.HIGHEST)
                  + ba_ref[...])


def _tc_mid(p, cp, wb, bb, wa, ba):
    return pl.pallas_call(
        _tc_mid_body,
        out_shape=(jax.ShapeDtypeStruct((NPAD, H), jnp.float32),
                   jax.ShapeDtypeStruct((NPAD, H), jnp.float32)),
    )(p, cp, wb, bb, wa, ba)


def _tc_final_body(p_ref, cp_ref, wb_ref, bb_ref, wf_ref, bf_ref, o_ref):
    h = _mean_h(p_ref, cp_ref, wb_ref, bb_ref)
    o_ref[...] = (jnp.dot(h, wf_ref[...], preferred_element_type=jnp.float32, precision=lax.Precision.HIGHEST)
                  + bf_ref[...])


def _tc_final(p, cp, wb, bb, wf, bf):
    return pl.pallas_call(
        _tc_final_body,
        out_shape=jax.ShapeDtypeStruct((NPAD, OUT), jnp.float32),
    )(p, cp, wb, bb, wf, bf)


def kernel(x, edge_index, edge_attr, w0a, b0a, w0b, b0b, w1a, b1a, w1b, b1b,
           w2a, b2a, w2b, b2b, wf, bf):
    src = edge_index[0]
    dst = edge_index[1]
    dsti = dst.reshape(NW, NBLK, CB)
    srci = src.reshape(NW, NBLK, CB)
    xpad = jnp.concatenate([x, jnp.zeros((NPAD - N, D), jnp.float32)], axis=0)
    zeros64 = jnp.zeros((NPAD, H), jnp.float32)
    zeros16 = jnp.zeros((NPAD, L), jnp.float32)

    a0, b0 = _tc_ab(xpad, w0a, b0a, D)
    t0, cntp = _make_edge0()(a0, b0, dsti, srci, zeros16)
    u = _tc_u(t0, w0b, b0b)
    h0 = _make_segmax()(u, dst, zeros64)
    a1, b1 = _tc_ab(h0, w1a, b1a, H)
    p1 = _make_edge_mean()(a1, b1, dsti, srci, zeros64)
    a2, b2 = _tc_mid(p1, cntp, w1b, b1b, w2a, b2a)
    p2 = _make_edge_mean()(a2, b2, dsti, srci, zeros64)
    out = _tc_final(p2, cntp, w2b, b2b, wf, bf)
    return out[:N]


# speedup vs baseline: 5.3261x; 1.7897x over previous
import functools

import jax
import jax.numpy as jnp
from jax import lax
from jax.experimental import pallas as pl
from jax.experimental.pallas import tpu as pltpu
from jax.experimental.pallas import tpu_sc as plsc

N = 10000
E = 320000
D = 128
H = 64
OUT = 4

NC = 2
NS = 16
NW = NC * NS
L = 16

NR = 320
NPAD = NW * NR
EPW = E // NW
CB = 80
NBLK = EPW // CB
ROWS_PER_TILE = NPAD // NS

CAP = 12800
SCHUNK = 20000
GB = 128
DBASE = CAP + GB + 16

_mesh = plsc.VectorSubcoreMesh(core_axis_name="c", subcore_axis_name="s")


def _wid():
    return lax.axis_index("s") * NC + lax.axis_index("c")


def _ew_add_relu(a_rows, b_rows):
    @plsc.parallel_loop(0, CB * H // L, unroll=8)
    def _(i):
        r = i >> 2
        col = (i & 3) * L
        av = a_rows[r, pl.ds(col, L)]
        bv = b_rows[r, pl.ds(col, L)]
        a_rows[r, pl.ds(col, L)] = jnp.maximum(av + bv, 0.0)


def _edge_pipeline(a_hbm, b_hbm, dstc, srcc, bufs, issue_cons, wait_cons,
                   extra=None):
    def issue_gather(j, p):
        ar, br, g, _ = bufs[p]
        pltpu.async_copy(a_hbm.at[dstc.at[j]], ar, g)
        pltpu.async_copy(b_hbm.at[srcc.at[j]], br, g)

    def wait_gather(j, p):
        ar, br, g, _ = bufs[p]
        pltpu.make_async_copy(a_hbm.at[dstc.at[j]], ar, g).wait()
        pltpu.make_async_copy(b_hbm.at[srcc.at[j]], br, g).wait()

    issue_gather(0, 0)

    def stage(j, p):
        q = 1 - p
        arq, _, _, stq = bufs[q]

        @pl.when(j >= 1)
        def _():
            wait_cons(j - 1, arq, stq)

        @pl.when(j + 1 < NBLK)
        def _():
            issue_gather(j + 1, q)

        ar, br, _, st = bufs[p]
        wait_gather(j, p)
        _ew_add_relu(ar, br)
        issue_cons(j, ar, st)
        if extra is not None:
            extra(j)

    def pair(jp, _):
        stage(2 * jp, 0)
        stage(2 * jp + 1, 1)
        return 0

    lax.fori_loop(0, NBLK // 2, pair, 0)
    stage(NBLK - 1, 0)
    wait_cons(NBLK - 1, bufs[0][0], bufs[0][3])


def _sc_edge0(a_hbm, b_hbm, dsti_hbm, srci_hbm, zeros16_hbm, t0_hbm, cntp_hbm,
              dstc, srcc, ar0, br0, ar1, br1, ones_v, cacc, g0, st0, g1, st1):
    c = lax.axis_index("c")
    s = lax.axis_index("s")
    wid = _wid()

    pltpu.sync_copy(dsti_hbm.at[wid], dstc)
    pltpu.sync_copy(srci_hbm.at[wid], srcc)
    pltpu.sync_copy(zeros16_hbm.at[pl.ds(s * ROWS_PER_TILE, ROWS_PER_TILE)],
                    cacc.at[pl.ds(s * ROWS_PER_TILE, ROWS_PER_TILE)])

    def fill_ones(i, _):
        ones_v[i, pl.ds(0, L)] = jnp.ones((L,), jnp.float32)
        return 0

    lax.fori_loop(0, CB, fill_ones, 0)
    plsc.subcore_barrier()

    def issue_cons(j, ar, st):
        pltpu.async_copy(ar, t0_hbm.at[pl.ds(wid * EPW + j * CB, CB)], st)

    def wait_cons(j, ar, st):
        pltpu.make_async_copy(ar, t0_hbm.at[pl.ds(wid * EPW + j * CB, CB)],
                              st).wait()

    def extra(j):
        pltpu.sync_copy(ones_v, cacc.at[dstc.at[j]], add=True)

    _edge_pipeline(a_hbm, b_hbm, dstc, srcc,
                   ((ar0, br0, g0, st0), (ar1, br1, g1, st1)),
                   issue_cons, wait_cons, extra)

    plsc.subcore_barrier()
    pltpu.sync_copy(cacc.at[pl.ds(s * ROWS_PER_TILE, ROWS_PER_TILE)],
                    cntp_hbm.at[c, pl.ds(s * ROWS_PER_TILE, ROWS_PER_TILE)])


def _sc_segmax(u_hbm, dstf_hbm, zeros_hbm, h0_hbm,
               accv, dchunk0, dchunk1, elist, urows0, urows1,
               dg0, dg1, ug0, ug1):
    wid = _wid()
    lo = wid * NR
    hi = lo + NR

    pltpu.sync_copy(zeros_hbm.at[pl.ds(0, NR)], accv)

    iota16 = lax.iota(jnp.int32, L)
    NCHUNK = E // SCHUNK
    dbufs = ((dchunk0, dg0), (dchunk1, dg1))

    def d_issue(cix, p):
        dc, dg = dbufs[p]
        pltpu.async_copy(dstf_hbm.at[pl.ds(cix * SCHUNK, SCHUNK)], dc, dg)

    def d_wait(cix, p):
        dc, dg = dbufs[p]
        pltpu.make_async_copy(dstf_hbm.at[pl.ds(cix * SCHUNK, SCHUNK)],
                              dc, dg).wait()

    d_issue(0, 0)

    def dstage(cix, p, off):
        @pl.when(cix + 1 < NCHUNK)
        def _():
            d_issue(cix + 1, 1 - p)

        d_wait(cix, p)
        dc, _ = dbufs[p]

        def scan16(i, off):
            dv = dc[pl.ds(i * L, L)]
            m = (dv >= lo) & (dv < hi)
            mi = jnp.where(m, 1, 0)
            cs = jnp.cumsum(mi)
            pos = off + cs - 1
            eid = iota16 + (cix * SCHUNK + i * L)
            plsc.store_scatter(elist, [pos], eid, mask=m)
            plsc.store_scatter(elist, [pos + DBASE], dv, mask=m)
            return off + jnp.max(cs)

        return lax.fori_loop(0, SCHUNK // L, scan16, off)

    def dpair(cp, off):
        off = dstage(2 * cp, 0, off)
        return dstage(2 * cp + 1, 1, off)

    off = lax.fori_loop(0, NCHUNK // 2, dpair, jnp.int32(0))

    def padz(i, _):
        pos = off + iota16 + i * L
        plsc.store_scatter(elist, [pos], jnp.zeros((L,), jnp.int32))
        return 0

    lax.fori_loop(0, GB // L + 1, padz, 0)

    nblks = (off + GB - 1) // GB
    ubufs = ((urows0, ug0), (urows1, ug1))

    def u_issue(b, p):
        ur, ug = ubufs[p]
        pltpu.async_copy(u_hbm.at[elist.at[pl.ds(b * GB, GB)]], ur, ug)

    def u_wait(b, p):
        ur, ug = ubufs[p]
        pltpu.make_async_copy(u_hbm.at[elist.at[pl.ds(b * GB, GB)]],
                              ur, ug).wait()

    @pl.when(nblks > 0)
    def _():
        u_issue(0, 0)

    def gstage(b, p):
        @pl.when(b + 1 < nblks)
        def _():
            u_issue(b + 1, 1 - p)

        u_wait(b, p)
        ur, _ = ubufs[p]

        def emax(q, _):
            dvec = elist[pl.ds(DBASE + b * GB + q * L, L)]
            for k2 in range(L):
                k = q * L + k2

                @pl.when(b * GB + k < off)
                def _():
                    dl = dvec[k2] - lo
                    for jj in range(H // L):
                        av = accv[dl, pl.ds(jj * L, L)]
                        uv = ur[k, pl.ds(jj * L, L)]
                        accv[dl, pl.ds(jj * L, L)] = jnp.maximum(av, uv)

            return 0

        lax.fori_loop(0, GB // L, emax, 0)

    def gpair(bp, _):
        @pl.when(2 * bp < nblks)
        def _():
            gstage(2 * bp, 0)

        @pl.when(2 * bp + 1 < nblks)
        def _():
            gstage(2 * bp + 1, 1)

        return 0

    lax.fori_loop(0, (nblks + 1) // 2, gpair, 0)
    pltpu.sync_copy(accv, h0_hbm.at[pl.ds(lo, NR)])


def _sc_edge_mean(a_hbm, b_hbm, dsti_hbm, srci_hbm, zeros_hbm, sump_hbm,
                  dstc, srcc, ar0, br0, ar1, br1, facc, g0, st0, g1, st1):
    c = lax.axis_index("c")
    s = lax.axis_index("s")
    wid = _wid()

    pltpu.sync_copy(dsti_hbm.at[wid], dstc)
    pltpu.sync_copy(srci_hbm.at[wid], srcc)
    pltpu.sync_copy(zeros_hbm.at[pl.ds(s * ROWS_PER_TILE, ROWS_PER_TILE)],
                    facc.at[pl.ds(s * ROWS_PER_TILE, ROWS_PER_TILE)])
    plsc.subcore_barrier()

    def issue_cons(j, ar, st):
        pltpu.async_copy(ar, facc.at[dstc.at[j]], st, add=True)

    def wait_cons(j, ar, st):
        pltpu.make_async_copy(ar, facc.at[dstc.at[j]], st).wait()

    _edge_pipeline(a_hbm, b_hbm, dstc, srcc,
                   ((ar0, br0, g0, st0), (ar1, br1, g1, st1)),
                   issue_cons, wait_cons)
    plsc.subcore_barrier()
    pltpu.sync_copy(facc.at[pl.ds(s * ROWS_PER_TILE, ROWS_PER_TILE)],
                    sump_hbm.at[c, pl.ds(s * ROWS_PER_TILE, ROWS_PER_TILE)])


def _make_edge0():
    return pl.kernel(
        _sc_edge0,
        out_type=(jax.ShapeDtypeStruct((E, H), jnp.float32),
                  jax.ShapeDtypeStruct((NC, NPAD, L), jnp.float32)),
        mesh=_mesh,
        compiler_params=pltpu.CompilerParams(use_tc_tiling_on_sc=False, needs_layout_passes=False),
        scratch_types=[
            pltpu.VMEM((NBLK, CB), jnp.int32),
            pltpu.VMEM((NBLK, CB), jnp.int32),
            pltpu.VMEM((CB, H), jnp.float32),
            pltpu.VMEM((CB, H), jnp.float32),
            pltpu.VMEM((CB, H), jnp.float32),
            pltpu.VMEM((CB, H), jnp.float32),
            pltpu.VMEM((CB, L), jnp.float32),
            pltpu.VMEM_SHARED((NPAD, L), jnp.float32),
            pltpu.SemaphoreType.DMA,
            pltpu.SemaphoreType.DMA,
            pltpu.SemaphoreType.DMA,
            pltpu.SemaphoreType.DMA,
        ],
    )


def _make_segmax():
    return pl.kernel(
        _sc_segmax,
        out_type=jax.ShapeDtypeStruct((NPAD, H), jnp.float32),
        mesh=_mesh,
        compiler_params=pltpu.CompilerParams(use_tc_tiling_on_sc=False, needs_layout_passes=False),
        scratch_types=[
            pltpu.VMEM((NR, H), jnp.float32),
            pltpu.VMEM((SCHUNK,), jnp.int32),
            pltpu.VMEM((SCHUNK,), jnp.int32),
            pltpu.VMEM((DBASE + CAP + GB + 16,), jnp.int32),
            pltpu.VMEM((GB, H), jnp.float32),
            pltpu.VMEM((GB, H), jnp.float32),
            pltpu.SemaphoreType.DMA,
            pltpu.SemaphoreType.DMA,
            pltpu.SemaphoreType.DMA,
            pltpu.SemaphoreType.DMA,
        ],
    )


def _make_edge_mean():
    return pl.kernel(
        _sc_edge_mean,
        out_type=jax.ShapeDtypeStruct((NC, NPAD, H), jnp.float32),
        mesh=_mesh,
        compiler_params=pltpu.CompilerParams(use_tc_tiling_on_sc=False, needs_layout_passes=False),
        scratch_types=[
            pltpu.VMEM((NBLK, CB), jnp.int32),
            pltpu.VMEM((NBLK, CB), jnp.int32),
            pltpu.VMEM((CB, H), jnp.float32),
            pltpu.VMEM((CB, H), jnp.float32),
            pltpu.VMEM((CB, H), jnp.float32),
            pltpu.VMEM((CB, H), jnp.float32),
            pltpu.VMEM_SHARED((NPAD, H), jnp.float32),
            pltpu.SemaphoreType.DMA,
            pltpu.SemaphoreType.DMA,
            pltpu.SemaphoreType.DMA,
            pltpu.SemaphoreType.DMA,
        ],
    )


def _tc_ab_body(din, x_ref, wa_ref, ba_ref, a_ref, b_ref):
    xv = x_ref[...]
    wtop = wa_ref[pl.ds(0, din), :]
    wbot = wa_ref[pl.ds(din, din), :]
    b_ref[...] = jnp.dot(xv, wbot, preferred_element_type=jnp.float32, precision=lax.Precision.HIGHEST)
    a_ref[...] = (jnp.dot(xv, wtop - wbot, preferred_element_type=jnp.float32, precision=lax.Precision.HIGHEST)
                  + ba_ref[...])


def _tc_ab(x, wa, ba, din):
    body = functools.partial(_tc_ab_body, din)
    return pl.pallas_call(
        body,
        out_shape=(jax.ShapeDtypeStruct((NPAD, H), jnp.float32),
                   jax.ShapeDtypeStruct((NPAD, H), jnp.float32)),
    )(x, wa, ba)


UB = 4000


def _tc_u_body(t_ref, w_ref, b_ref, u_ref):
    u_ref[...] = (jnp.dot(t_ref[...], w_ref[...],
                          preferred_element_type=jnp.float32, precision=lax.Precision.HIGHEST) + b_ref[...])


def _tc_u(t0, w, b):
    return pl.pallas_call(
        _tc_u_body,
        grid=(E // UB,),
        in_specs=[
            pl.BlockSpec((UB, H), lambda i: (i, 0)),
            pl.BlockSpec((H, H), lambda i: (0, 0)),
            pl.BlockSpec((H,), lambda i: (0,)),
        ],
        out_specs=pl.BlockSpec((UB, H), lambda i: (i, 0)),
        out_shape=jax.ShapeDtypeStruct((E, H), jnp.float32),
    )(t0, w, b)


def _mean_h(p_ref, cp_ref, wb_ref, bb_ref):
    sums = p_ref[0] + p_ref[1]
    cnt = cp_ref[0, :, pl.ds(0, 1)] + cp_ref[1, :, pl.ds(0, 1)]
    mean = sums / jnp.maximum(cnt, 1.0)
    h = jnp.dot(mean, wb_ref[...], preferred_element_type=jnp.float32, precision=lax.Precision.HIGHEST) + bb_ref[...]
    return jnp.where(cnt > 0.0, h, 0.0)


def _tc_mid_body(p_ref, cp_ref, wb_ref, bb_ref, wa_ref, ba_ref, a_ref, b_ref):
    h = jnp.maximum(_mean_h(p_ref, cp_ref, wb_ref, bb_ref), 0.0)
    wtop = wa_ref[pl.ds(0, H), :]
    wbot = wa_ref[pl.ds(H, H), :]
    b_ref[...] = jnp.dot(h, wbot, preferred_element_type=jnp.float32, precision=lax.Precision.HIGHEST)
    a_ref[...] = (jnp.dot(h, wtop - wbot, preferred_element_type=jnp.float32, precision=lax.Precision.HIGHEST)
                  + ba_ref[...])


def _tc_mid(p, cp, wb, bb, wa, ba):
    return pl.pallas_call(
        _tc_mid_body,
        out_shape=(jax.ShapeDtypeStruct((NPAD, H), jnp.float32),
                   jax.ShapeDtypeStruct((NPAD, H), jnp.float32)),
    )(p, cp, wb, bb, wa, ba)


def _tc_final_body(p_ref, cp_ref, wb_ref, bb_ref, wf_ref, bf_ref, o_ref):
    h = _mean_h(p_ref, cp_ref, wb_ref, bb_ref)
    o_ref[...] = (jnp.dot(h, wf_ref[...], preferred_element_type=jnp.float32, precision=lax.Precision.HIGHEST)
                  + bf_ref[...])


def _tc_final(p, cp, wb, bb, wf, bf):
    return pl.pallas_call(
        _tc_final_body,
        out_shape=jax.ShapeDtypeStruct((NPAD, OUT), jnp.float32),
    )(p, cp, wb, bb, wf, bf)


def kernel(x, edge_index, edge_attr, w0a, b0a, w0b, b0b, w1a, b1a, w1b, b1b,
           w2a, b2a, w2b, b2b, wf, bf):
    src = edge_index[0]
    dst = edge_index[1]
    dsti = dst.reshape(NW, NBLK, CB)
    srci = src.reshape(NW, NBLK, CB)
    xpad = jnp.concatenate([x, jnp.zeros((NPAD - N, D), jnp.float32)], axis=0)
    zeros64 = jnp.zeros((NPAD, H), jnp.float32)
    zeros16 = jnp.zeros((NPAD, L), jnp.float32)

    a0, b0 = _tc_ab(xpad, w0a, b0a, D)
    t0, cntp = _make_edge0()(a0, b0, dsti, srci, zeros16)
    u = _tc_u(t0, w0b, b0b)
    h0 = _make_segmax()(u, dst, zeros64)
    a1, b1 = _tc_ab(h0, w1a, b1a, H)
    p1 = _make_edge_mean()(a1, b1, dsti, srci, zeros64)
    a2, b2 = _tc_mid(p1, cntp, w1b, b1b, w2a, b2a)
    p2 = _make_edge_mean()(a2, b2, dsti, srci, zeros64)
    out = _tc_final(p2, cntp, w2b, b2b, wf, bf)
    return out[:N]
